# baseline (speedup 1.0000x reference)
import jax
import jax.numpy as jnp
from jax import lax
from jax.experimental import pallas as pl
from jax.experimental.pallas import tpu as pltpu
from jax.experimental.pallas import tpu_sc as plsc

B, N, S, SO, K, C = 4, 400, 12, 12, 32, 6
NW = 32
TPB = NW // B
NPT = N // TPB
NCH = N // 16

OB1 = 8
OB2 = 20
OW = 32
OA = 176
OD1 = 320
OD2 = 608
WLEN = 688

F32 = jnp.float32
I32 = jnp.int32
TRIU_PAD = (K * K - K * (K - 1) // 2) * 1e-5


def _splat_i(x):
    return jnp.full((16,), x, I32)


def _bf16r(x):
    y = plsc.bitcast(x, I32)
    lsb = jax.lax.shift_right_logical(y, 16) & 1
    r = (y + (32767 + lsb)) & jnp.int32(-65536)
    return plsc.bitcast(r, F32)


def _body(inp_hbm, adj_hbm, wgt_hbm, out_hbm, part_hbm,
          inp_v, wh_v, a1_v, a2_v, h2_v, adj_v, wgt_v, outbuf,
          scr_att, scr_mask, scr_wct, scr_n, scr_b1, scr_part):
    wid = lax.axis_index("s") * 2 + lax.axis_index("c")
    b = wid // TPB
    t8 = wid % TPB
    i0 = t8 * NPT

    pltpu.sync_copy(inp_hbm.at[pl.ds(b * (S * N), S * N)], inp_v)
    pltpu.sync_copy(adj_hbm.at[pl.ds(b * (N * K) + i0 * K, NPT * K)], adj_v)
    pltpu.sync_copy(wgt_hbm, wgt_v)

    for t in range((WLEN - OW) // 16):
        off = OW + 16 * t
        wgt_v[pl.ds(off, 16)] = _bf16r(wgt_v[pl.ds(off, 16)])

    iota = lax.iota(I32, 16)
    iota400 = iota * N
    lane_lt12 = iota < SO
    zeros16 = jnp.zeros((16,), F32)

    def wsplat(idx):
        return plsc.load_gather(wgt_v, [_splat_i(idx)])

    def chunk_body(ch, whsum):
        base = ch * 16
        xs = [_bf16r(plsc.load_gather(inp_v, [iota + (s * N + base)]))
              for s in range(S)]
        whs = []
        for so in range(SO):
            acc = xs[0] * wsplat(OW + so)
            for s in range(1, S):
                acc = acc + xs[s] * wsplat(OW + s * SO + so)
            whsum = whsum + acc
            accr = _bf16r(acc)
            plsc.store_scatter(wh_v, [iota + (so * N + base)], accr)
            whs.append(accr)
        for c in range(C):
            a1 = whs[0] * wsplat(OA + c)
            a2 = whs[0] * wsplat(OA + SO * C + c)
            for so in range(1, SO):
                a1 = a1 + whs[so] * wsplat(OA + so * C + c)
                a2 = a2 + whs[so] * wsplat(OA + (SO + so) * C + c)
            plsc.store_scatter(a1_v, [iota + (c * N + base)], a1)
            plsc.store_scatter(a2_v, [iota + (c * N + base)], a2)
        for f in range(SO):
            h2 = whs[0] * wsplat(OD1 + SO * SO + f)
            for d in range(1, SO):
                h2 = h2 + whs[d] * wsplat(OD1 + (SO + d) * SO + f)
            plsc.store_scatter(h2_v, [iota + (f * N + base)], h2)
        return whsum

    whsum = plsc.parallel_loop(0, NCH, carry=zeros16)(chunk_body)
    wh_sum = jnp.sum(whsum) * jnp.where(t8 == 0, 1.0, 0.0).astype(F32)

    wd1A = [plsc.load_gather(wgt_v, [iota + (OD1 + d * SO)]) for d in range(SO)]
    bd1_v = plsc.load_gather(wgt_v, [iota + OB1])

    def node_body(li, carry):
        att_sum, dist_sum, cl_sum = carry
        i = i0 + li
        abase = li * K
        idxs = (plsc.load_gather(adj_v, [iota + abase]),
                plsc.load_gather(adj_v, [iota + (abase + 16)]))

        att = [[None, None] for _ in range(C)]
        atv = zeros16
        for c in range(C):
            a1c = plsc.load_gather(a1_v, [_splat_i(c * N + i)])
            for j in range(2):
                v = plsc.load_gather(a2_v, [idxs[j] + (c * N)]) + a1c
                att[c][j] = v
                atv = atv + v
                scr_att[pl.ds(c * 32 + j * 16, 16)] = v
        att_sum = att_sum + jnp.sum(atv)

        es = [[jnp.exp(att[c][j]) for j in range(2)] for c in range(C)]
        for c in range(C):
            invv = 1.0 / jnp.full((16,), jnp.sum(es[c][0]) + jnp.sum(es[c][1]), F32)
            scr_mask[pl.ds(c * 32, 16)] = _bf16r(es[c][0] * invv)
            scr_mask[pl.ds(c * 32 + 16, 16)] = _bf16r(es[c][1] * invv)

        b1 = bd1_v
        for d in range(SO):
            b1 = b1 + plsc.load_gather(wh_v, [_splat_i(d * N + i)]) * wd1A[d]
        scr_b1[pl.ds(8, 16)] = b1
        b1s = [plsc.load_gather(scr_b1, [_splat_i(8 + f)]) for f in range(SO)]

        hvec = [[None, None] for _ in range(SO)]
        for f in range(SO):
            for j in range(2):
                acc = plsc.load_gather(h2_v, [idxs[j] + (f * N)]) + b1s[f]
                hvec[f][j] = _bf16r(jnp.where(acc >= 0, acc, 0.01 * acc))
        wct = [[None, None] for _ in range(C)]
        for g in range(C):
            wcol = [wsplat(OD2 + f * C + g) for f in range(SO)]
            bg = wsplat(OB2 + g)
            for j in range(2):
                acc = hvec[0][j] * wcol[0]
                for f in range(1, SO):
                    acc = acc + hvec[f][j] * wcol[f]
                acc = acc + bg
                v = jnp.where(acc >= 0, acc, 0.01 * acc)
                wct[g][j] = v
                scr_wct[pl.ds(g * 32 + j * 16, 16)] = _bf16r(v)
        nv = []
        for j in range(2):
            nn = wct[0][j] * wct[0][j]
            for g in range(1, C):
                nn = nn + wct[g][j] * wct[g][j]
            nv.append(nn)
        wctr = [[_bf16r(wct[g][j]) for j in range(2)] for g in range(C)]
        scr_n[pl.ds(0, 16)] = nv[0]
        scr_n[pl.ds(16, 16)] = nv[1]

        def splats(k):
            return ([plsc.load_gather(scr_att, [_splat_i(c * 32 + k)]) for c in range(C)],
                    [plsc.load_gather(scr_wct, [_splat_i(g * 32 + k)]) for g in range(C)],
                    plsc.load_gather(scr_n, [_splat_i(k)]))

        z6 = tuple(zeros16 for _ in range(C))

        def mk_body(spec):
            def body(k, pc):
                awn = splats(k)
                pds, pis, dsv = pc
                pds_l, pis_l = list(pds), list(pis)
                for j, msk in spec:
                    s = [awn[0][c] * att[c][j] for c in range(C)]
                    m = s[0]
                    for c in range(1, C):
                        m = jnp.maximum(m, s[c])
                    e = [jnp.exp(s[c] - m) for c in range(C)]
                    den = e[0]
                    for c in range(1, C):
                        den = den + e[c]
                    inv = 1.0 / den
                    dot = awn[1][0] * wctr[0][j]
                    for g in range(1, C):
                        dot = dot + awn[1][g] * wctr[g][j]
                    dist = (awn[2] + nv[j]) - 2.0 * dot
                    if msk:
                        mfac = jnp.where((iota + j * 16) > k, 1.0, 0.0).astype(F32)
                        dist = dist * mfac
                    dsv = dsv + dist
                    for c in range(C):
                        p = e[c] * inv
                        mp = jnp.maximum(p, 1e-5)
                        pds_l[c] = pds_l[c] + dist * p
                        pis_l[c] = pis_l[c] + (mfac * mp if msk else mp)
                return (tuple(pds_l), tuple(pis_l), dsv)
            return body

        pc = lax.fori_loop(0, 16, mk_body(((0, True), (1, False))), (z6, z6, zeros16))
        pds, pis, dsv = lax.fori_loop(16, K - 1, mk_body(((1, True),)), pc)
        for c in range(C):
            num = jnp.full((16,), jnp.sum(pds[c]), F32)
            den = jnp.full((16,), jnp.sum(pis[c]) + TRIU_PAD, F32)
            cl_sum = cl_sum + num / den
        ddiag = zeros16
        for j in range(2):
            dj = wctr[0][j] * wctr[0][j]
            for g in range(1, C):
                dj = dj + wctr[g][j] * wctr[g][j]
            ddiag = ddiag + (nv[j] - dj)
        dist_sum = dist_sum + 2.0 * (jnp.sum(dsv) + jnp.sum(ddiag))

        def out_body(k, oacc):
            idxk = plsc.load_gather(adj_v, [_splat_i(abase + k)])
            ridx = jnp.where(lane_lt12, iota400 + idxk, 0)
            row = jnp.where(lane_lt12, plsc.load_gather(wh_v, [ridx]), 0.0)
            o = list(oacc)
            for c in range(C):
                mk = plsc.load_gather(scr_mask, [_splat_i(c * 32 + k)])
                o[c] = o[c] + mk * row
            return tuple(o)

        oacc = lax.fori_loop(0, K, out_body, z6)

        for c in range(C):
            sidx = jnp.where(lane_lt12, iota + (li * C + c) * SO, NPT * C * SO)
            plsc.store_scatter(outbuf, [sidx], oacc[c])
        return (att_sum, dist_sum, cl_sum)

    zf = jnp.zeros((), F32)
    att_sum, dist_sum, cl_sum = lax.fori_loop(
        0, NPT, node_body, (zf, zf, zeros16))

    pairs = [(c1, c2) for c1 in range(C) for c2 in range(c1 + 1, C)]
    ccl_v = zeros16
    for nc in range((NPT + 15) // 16):
        nid = iota + nc * 16
        valid = nid < NPT
        nbase = jnp.where(valid, nid, 0) * (C * SO)
        on = [zeros16 for _ in range(C)]
        dd = {pr: zeros16 for pr in pairs}
        for so in range(SO):
            g = [plsc.load_gather(outbuf, [nbase + (c * SO + so)]) for c in range(C)]
            r = [_bf16r(g[c]) for c in range(C)]
            for c in range(C):
                on[c] = on[c] + g[c] * g[c]
            for (c1, c2) in pairs:
                dd[(c1, c2)] = dd[(c1, c2)] + r[c1] * r[c2]
        for (c1, c2) in pairs:
            t = jnp.maximum(5.0 - (on[c1] + on[c2] - 2.0 * dd[(c1, c2)]), 0.0)
            ccl_v = ccl_v + jnp.where(valid, t * t, 0.0)
    ccl_sum = jnp.sum(ccl_v)

    pv = (jnp.where(iota == 0, att_sum, 0.0)
          + jnp.where(iota == 1, dist_sum, 0.0)
          + jnp.where(iota == 2, cl_sum, 0.0)
          + jnp.where(iota == 3, ccl_sum, 0.0)
          + jnp.where(iota == 4, wh_sum, 0.0))
    scr_part[pl.ds(0, 16)] = pv
    pltpu.sync_copy(scr_part, part_hbm.at[pl.ds(wid * 16, 16)])
    pltpu.sync_copy(outbuf.at[pl.ds(0, NPT * C * SO)],
                    out_hbm.at[pl.ds(b * (N * C * SO) + i0 * (C * SO), NPT * C * SO)])


@jax.jit
def kernel(fushed_features, input_data, adj_idx, W_w, W_a, Wd1, bd1, Wd2, bd2):
    del fushed_features

    def r16(a):
        return a.astype(jnp.bfloat16).astype(F32)

    inp_flat = r16(jnp.swapaxes(input_data, 1, 2)).reshape(-1)
    adj_flat = adj_idx.astype(jnp.int32).reshape(-1)
    z8 = jnp.zeros((8,), F32)
    wgt = jnp.concatenate([z8, bd1, bd2, jnp.zeros((OW - 26,), F32),
                           r16(W_w).reshape(-1), r16(W_a).reshape(-1),
                           r16(Wd1).reshape(-1), r16(Wd2).reshape(-1),
                           jnp.zeros((WLEN - OD2 - 72,), F32)])

    mesh = plsc.VectorSubcoreMesh(core_axis_name="c", subcore_axis_name="s",
                                  num_cores=2, num_subcores=16)
    call = pl.kernel(
        _body,
        out_type=(jax.ShapeDtypeStruct((B * N * C * SO,), F32),
                  jax.ShapeDtypeStruct((NW * 16,), F32)),
        mesh=mesh,
        compiler_params=pltpu.CompilerParams(needs_layout_passes=False),
        scratch_types=[
            pltpu.VMEM((S * N,), F32),
            pltpu.VMEM((SO * N,), F32),
            pltpu.VMEM((C * N,), F32),
            pltpu.VMEM((C * N,), F32),
            pltpu.VMEM((SO * N,), F32),
            pltpu.VMEM((NPT * K,), jnp.int32),
            pltpu.VMEM((WLEN,), F32),
            pltpu.VMEM((NPT * C * SO + 8,), F32),
            pltpu.VMEM((C * K,), F32),
            pltpu.VMEM((C * K,), F32),
            pltpu.VMEM((C * K,), F32),
            pltpu.VMEM((K,), F32),
            pltpu.VMEM((24,), F32),
            pltpu.VMEM((16,), F32),
        ],
    )
    out_flat, parts = call(inp_flat, adj_flat, wgt)

    output_data = out_flat.reshape(B, N, C, SO)
    p = parts.reshape(NW, 16)
    cluster_loss = p[:, 2].sum() / (B * N)
    cluster_center_loss = p[:, 3].sum() / (B * N)
    wh_mean = p[:, 4].sum() / (B * N * SO)
    att_mean = p[:, 0].sum() / (B * N * K * C)
    dist_mean = p[:, 1].sum() / (B * N * K * K)
    return (output_data, cluster_loss, cluster_center_loss,
            wh_mean, att_mean, dist_mean)


# speedup vs baseline: 1.0197x; 1.0197x over previous
import jax
import jax.numpy as jnp
from jax import lax
from jax.experimental import pallas as pl
from jax.experimental.pallas import tpu as pltpu
from jax.experimental.pallas import tpu_sc as plsc

B, N, S, SO, K, C = 4, 400, 12, 12, 32, 6
NW = 32
TPB = NW // B
NPT = N // TPB
NCH = N // 16

OB1 = 8
OB2 = 20
OW = 32
OA = 176
OD1 = 320
OD2 = 608
WLEN = 688

F32 = jnp.float32
I32 = jnp.int32
TRIU_PAD = (K * K - K * (K - 1) // 2) * 1e-5


def _splat_i(x):
    return jnp.full((16,), x, I32)


def _bf16r(x):
    y = plsc.bitcast(x, I32)
    lsb = jax.lax.shift_right_logical(y, 16) & 1
    r = (y + (32767 + lsb)) & jnp.int32(-65536)
    return plsc.bitcast(r, F32)


def _body(inp_hbm, adj_hbm, wgt_hbm, out_hbm, part_hbm,
          inp_v, wh_v, a1_v, a2_v, h2_v, adj_v, wgt_v, outbuf,
          scr_att, scr_mask, scr_wct, scr_n, scr_b1, scr_part):
    wid = lax.axis_index("s") * 2 + lax.axis_index("c")
    b = wid // TPB
    t8 = wid % TPB
    i0 = t8 * NPT

    pltpu.sync_copy(inp_hbm.at[pl.ds(b * (S * N), S * N)], inp_v)
    pltpu.sync_copy(adj_hbm.at[pl.ds(b * (N * K) + i0 * K, NPT * K)], adj_v)
    pltpu.sync_copy(wgt_hbm, wgt_v)

    for t in range((WLEN - OW) // 16):
        off = OW + 16 * t
        wgt_v[pl.ds(off, 16)] = _bf16r(wgt_v[pl.ds(off, 16)])

    iota = lax.iota(I32, 16)
    iota400 = iota * N
    lane_lt12 = iota < SO
    zeros16 = jnp.zeros((16,), F32)

    def wsplat(idx):
        return plsc.load_gather(wgt_v, [_splat_i(idx)])

    def chunk_body(ch, whsum):
        base = ch * 16
        xs = [_bf16r(plsc.load_gather(inp_v, [iota + (s * N + base)]))
              for s in range(S)]
        whs = []
        for so in range(SO):
            acc = xs[0] * wsplat(OW + so)
            for s in range(1, S):
                acc = acc + xs[s] * wsplat(OW + s * SO + so)
            whsum = whsum + acc
            accr = _bf16r(acc)
            plsc.store_scatter(wh_v, [iota + (so * N + base)], accr)
            whs.append(accr)
        for c in range(C):
            a1 = whs[0] * wsplat(OA + c)
            a2 = whs[0] * wsplat(OA + SO * C + c)
            for so in range(1, SO):
                a1 = a1 + whs[so] * wsplat(OA + so * C + c)
                a2 = a2 + whs[so] * wsplat(OA + (SO + so) * C + c)
            plsc.store_scatter(a1_v, [iota + (c * N + base)], a1)
            plsc.store_scatter(a2_v, [iota + (c * N + base)], a2)
        for f in range(SO):
            h2 = whs[0] * wsplat(OD1 + SO * SO + f)
            for d in range(1, SO):
                h2 = h2 + whs[d] * wsplat(OD1 + (SO + d) * SO + f)
            plsc.store_scatter(h2_v, [iota + (f * N + base)], h2)
        return whsum

    whsum = plsc.parallel_loop(0, NCH, carry=zeros16)(chunk_body)
    wh_sum = jnp.sum(whsum) * jnp.where(t8 == 0, 1.0, 0.0).astype(F32)

    wd1A = [plsc.load_gather(wgt_v, [iota + (OD1 + d * SO)]) for d in range(SO)]
    bd1_v = plsc.load_gather(wgt_v, [iota + OB1])

    def node_body(li, carry):
        att_sum, dist_sum, cl_sum, ccl_sum = carry
        i = i0 + li
        abase = li * K
        idxs = (plsc.load_gather(adj_v, [iota + abase]),
                plsc.load_gather(adj_v, [iota + (abase + 16)]))

        att = [[None, None] for _ in range(C)]
        atv = zeros16
        for c in range(C):
            a1c = plsc.load_gather(a1_v, [_splat_i(c * N + i)])
            for j in range(2):
                v = plsc.load_gather(a2_v, [idxs[j] + (c * N)]) + a1c
                att[c][j] = v
                atv = atv + v
                scr_att[pl.ds(c * 32 + j * 16, 16)] = v
        att_sum = att_sum + jnp.sum(atv)

        es = [[jnp.exp(att[c][j]) for j in range(2)] for c in range(C)]
        for c in range(C):
            invv = 1.0 / jnp.full((16,), jnp.sum(es[c][0]) + jnp.sum(es[c][1]), F32)
            scr_mask[pl.ds(c * 32, 16)] = _bf16r(es[c][0] * invv)
            scr_mask[pl.ds(c * 32 + 16, 16)] = _bf16r(es[c][1] * invv)

        b1 = bd1_v
        for d in range(SO):
            b1 = b1 + plsc.load_gather(wh_v, [_splat_i(d * N + i)]) * wd1A[d]
        scr_b1[pl.ds(8, 16)] = b1
        b1s = [plsc.load_gather(scr_b1, [_splat_i(8 + f)]) for f in range(SO)]

        hvec = [[None, None] for _ in range(SO)]
        for f in range(SO):
            for j in range(2):
                acc = plsc.load_gather(h2_v, [idxs[j] + (f * N)]) + b1s[f]
                hvec[f][j] = _bf16r(jnp.where(acc >= 0, acc, 0.01 * acc))
        wct = [[None, None] for _ in range(C)]
        for g in range(C):
            wcol = [wsplat(OD2 + f * C + g) for f in range(SO)]
            bg = wsplat(OB2 + g)
            for j in range(2):
                acc = hvec[0][j] * wcol[0]
                for f in range(1, SO):
                    acc = acc + hvec[f][j] * wcol[f]
                acc = acc + bg
                v = jnp.where(acc >= 0, acc, 0.01 * acc)
                wct[g][j] = v
                scr_wct[pl.ds(g * 32 + j * 16, 16)] = _bf16r(v)
        nv = []
        for j in range(2):
            nn = wct[0][j] * wct[0][j]
            for g in range(1, C):
                nn = nn + wct[g][j] * wct[g][j]
            nv.append(nn)
        wctr = [[_bf16r(wct[g][j]) for j in range(2)] for g in range(C)]
        scr_n[pl.ds(0, 16)] = nv[0]
        scr_n[pl.ds(16, 16)] = nv[1]

        def splats(k):
            return ([plsc.load_gather(scr_att, [_splat_i(c * 32 + k)]) for c in range(C)],
                    [plsc.load_gather(scr_wct, [_splat_i(g * 32 + k)]) for g in range(C)],
                    plsc.load_gather(scr_n, [_splat_i(k)]))

        z6 = tuple(zeros16 for _ in range(C))

        def mk_body(spec):
            def body(k, pc):
                awn = splats(k)
                pds, pis, dsv = pc
                pds_l, pis_l = list(pds), list(pis)
                for j, msk in spec:
                    s = [awn[0][c] * att[c][j] for c in range(C)]
                    m = s[0]
                    for c in range(1, C):
                        m = jnp.maximum(m, s[c])
                    e = [jnp.exp(s[c] - m) for c in range(C)]
                    den = e[0]
                    for c in range(1, C):
                        den = den + e[c]
                    inv = 1.0 / den
                    dot = awn[1][0] * wctr[0][j]
                    for g in range(1, C):
                        dot = dot + awn[1][g] * wctr[g][j]
                    dist = (awn[2] + nv[j]) - 2.0 * dot
                    if msk:
                        mfac = jnp.where((iota + j * 16) > k, 1.0, 0.0).astype(F32)
                        dist = dist * mfac
                    dsv = dsv + dist
                    for c in range(C):
                        p = e[c] * inv
                        mp = jnp.maximum(p, 1e-5)
                        pds_l[c] = pds_l[c] + dist * p
                        pis_l[c] = pis_l[c] + (mfac * mp if msk else mp)
                return (tuple(pds_l), tuple(pis_l), dsv)
            return body

        pc = lax.fori_loop(0, 16, mk_body(((0, True), (1, False))), (z6, z6, zeros16))
        pds, pis, dsv = lax.fori_loop(16, K - 1, mk_body(((1, True),)), pc)
        for c in range(C):
            num = jnp.full((16,), jnp.sum(pds[c]), F32)
            den = jnp.full((16,), jnp.sum(pis[c]) + TRIU_PAD, F32)
            cl_sum = cl_sum + num / den
        ddiag = zeros16
        for j in range(2):
            dj = wctr[0][j] * wctr[0][j]
            for g in range(1, C):
                dj = dj + wctr[g][j] * wctr[g][j]
            ddiag = ddiag + (nv[j] - dj)
        dist_sum = dist_sum + 2.0 * (jnp.sum(dsv) + jnp.sum(ddiag))

        def out_body(k, oacc):
            idxk = plsc.load_gather(adj_v, [_splat_i(abase + k)])
            ridx = jnp.where(lane_lt12, iota400 + idxk, 0)
            row = jnp.where(lane_lt12, plsc.load_gather(wh_v, [ridx]), 0.0)
            o = list(oacc)
            for c in range(C):
                mk = plsc.load_gather(scr_mask, [_splat_i(c * 32 + k)])
                o[c] = o[c] + mk * row
            return tuple(o)

        oacc = lax.fori_loop(0, K, out_body, z6)

        on = [jnp.sum(oacc[c] * oacc[c]) for c in range(C)]
        oaccr = [_bf16r(oacc[c]) for c in range(C)]
        for c in range(C):
            sidx = jnp.where(lane_lt12, iota + (li * C + c) * SO, NPT * C * SO)
            plsc.store_scatter(outbuf, [sidx], oacc[c])
        for c1 in range(C):
            for c2 in range(c1 + 1, C):
                dd = jnp.sum(oaccr[c1] * oaccr[c2])
                t = jnp.maximum(5.0 - (on[c1] + on[c2] - 2.0 * dd), 0.0)
                ccl_sum = ccl_sum + t * t
        return (att_sum, dist_sum, cl_sum, ccl_sum)

    zf = jnp.zeros((), F32)
    att_sum, dist_sum, cl_sum, ccl_sum = lax.fori_loop(
        0, NPT, node_body, (zf, zf, zeros16, zf))

    pv = (jnp.where(iota == 0, att_sum, 0.0)
          + jnp.where(iota == 1, dist_sum, 0.0)
          + jnp.where(iota == 2, cl_sum, 0.0)
          + jnp.where(iota == 3, ccl_sum, 0.0)
          + jnp.where(iota == 4, wh_sum, 0.0))
    scr_part[pl.ds(0, 16)] = pv
    pltpu.sync_copy(scr_part, part_hbm.at[pl.ds(wid * 16, 16)])
    pltpu.sync_copy(outbuf.at[pl.ds(0, NPT * C * SO)],
                    out_hbm.at[pl.ds(b * (N * C * SO) + i0 * (C * SO), NPT * C * SO)])


@jax.jit
def kernel(fushed_features, input_data, adj_idx, W_w, W_a, Wd1, bd1, Wd2, bd2):
    del fushed_features

    def r16(a):
        return a.astype(jnp.bfloat16).astype(F32)

    inp_flat = r16(jnp.swapaxes(input_data, 1, 2)).reshape(-1)
    adj_flat = adj_idx.astype(jnp.int32).reshape(-1)
    z8 = jnp.zeros((8,), F32)
    wgt = jnp.concatenate([z8, bd1, bd2, jnp.zeros((OW - 26,), F32),
                           r16(W_w).reshape(-1), r16(W_a).reshape(-1),
                           r16(Wd1).reshape(-1), r16(Wd2).reshape(-1),
                           jnp.zeros((WLEN - OD2 - 72,), F32)])

    mesh = plsc.VectorSubcoreMesh(core_axis_name="c", subcore_axis_name="s",
                                  num_cores=2, num_subcores=16)
    call = pl.kernel(
        _body,
        out_type=(jax.ShapeDtypeStruct((B * N * C * SO,), F32),
                  jax.ShapeDtypeStruct((NW * 16,), F32)),
        mesh=mesh,
        compiler_params=pltpu.CompilerParams(needs_layout_passes=False),
        scratch_types=[
            pltpu.VMEM((S * N,), F32),
            pltpu.VMEM((SO * N,), F32),
            pltpu.VMEM((C * N,), F32),
            pltpu.VMEM((C * N,), F32),
            pltpu.VMEM((SO * N,), F32),
            pltpu.VMEM((NPT * K,), jnp.int32),
            pltpu.VMEM((WLEN,), F32),
            pltpu.VMEM((NPT * C * SO + 8,), F32),
            pltpu.VMEM((C * K,), F32),
            pltpu.VMEM((C * K,), F32),
            pltpu.VMEM((C * K,), F32),
            pltpu.VMEM((K,), F32),
            pltpu.VMEM((24,), F32),
            pltpu.VMEM((16,), F32),
        ],
    )
    out_flat, parts = call(inp_flat, adj_flat, wgt)

    output_data = out_flat.reshape(B, N, C, SO)
    p = parts.reshape(NW, 16)
    cluster_loss = p[:, 2].sum() / (B * N)
    cluster_center_loss = p[:, 3].sum() / (B * N)
    wh_mean = p[:, 4].sum() / (B * N * SO)
    att_mean = p[:, 0].sum() / (B * N * K * C)
    dist_mean = p[:, 1].sum() / (B * N * K * K)
    return (output_data, cluster_loss, cluster_center_loss,
            wh_mean, att_mean, dist_mean)
